# Initial kernel scaffold; baseline (speedup 1.0000x reference)
import jax
import jax.numpy as jnp
from jax.experimental import pallas as pl


def kernel(hidden_states, router_w, gate_w, up_w, down_w, shared_gate_w, shared_up_w, shared_down_w):
    raise NotImplementedError("write your pallas kernel here")


# speedup vs baseline: 1.4426x; 1.4426x over previous
import functools

import jax
import jax.numpy as jnp
from jax import lax
from jax.experimental import pallas as pl
from jax.experimental.pallas import tpu as pltpu

HIDDEN = 1024
FF = 2048
E = 8
TOK = 2048
FFB = 512
NFF = FF // FFB


def _nt(a, b):
    return lax.dot_general(a, b, (((1,), (1,)), ((), ())),
                           preferred_element_type=jnp.float32)


def _router_scores(x, rw):
    logits = _nt(x, rw)
    ii = lax.broadcasted_iota(jnp.int32, (TOK, E), 1)
    m1 = jnp.max(logits, axis=1, keepdims=True)
    idx1 = jnp.min(jnp.where(logits == m1, ii, E), axis=1, keepdims=True)
    mask1 = ii == idx1
    rest = jnp.where(mask1, -jnp.inf, logits)
    m2 = jnp.max(rest, axis=1, keepdims=True)
    idx2 = jnp.min(jnp.where(rest == m2, ii, E), axis=1, keepdims=True)
    topmask = mask1 | (ii == idx2)
    return jnp.where(topmask, jax.nn.sigmoid(logits), 0.0)


def _shared_body(hs_ref, rw_ref, sg_ref, su_ref, sd_ref,
                 scores_ref, base_ref, acc_ref):
    f = pl.program_id(0)
    x = hs_ref[...]

    @pl.when(f == 0)
    def _():
        scores_ref[...] = _router_scores(x, rw_ref[...])

    h = jax.nn.silu(_nt(x, sg_ref[...])) * _nt(x, su_ref[...])
    part = _nt(h, sd_ref[...])

    @pl.when(f == 0)
    def _():
        acc_ref[...] = part

    @pl.when(f > 0)
    def _():
        acc_ref[...] += part

    @pl.when(f == NFF - 1)
    def _():
        base_ref[...] = acc_ref[...]


def _experts_body(hs_ref, scores_ref, base_ref, wg_ref, wu_ref, wd_ref,
                  out_ref, acc_ref):
    e = pl.program_id(0)
    f = pl.program_id(1)
    x = hs_ref[...]
    h = jax.nn.silu(_nt(x, wg_ref[0])) * _nt(x, wu_ref[0])
    part = _nt(h, wd_ref[0])
    ii = lax.broadcasted_iota(jnp.int32, (TOK, E), 1)
    sc = jnp.sum(jnp.where(ii == e, scores_ref[...], 0.0), axis=1,
                 keepdims=True)
    contrib = part * sc

    first = jnp.logical_and(e == 0, f == 0)

    @pl.when(first)
    def _():
        acc_ref[...] = base_ref[...] + contrib

    @pl.when(jnp.logical_not(first))
    def _():
        acc_ref[...] += contrib

    @pl.when(jnp.logical_and(e == E - 1, f == NFF - 1))
    def _():
        out_ref[...] = acc_ref[...]


def kernel(hidden_states, router_w, gate_w, up_w, down_w,
           shared_gate_w, shared_up_w, shared_down_w):
    hs = hidden_states.reshape(-1, HIDDEN)

    scores, base = pl.pallas_call(
        _shared_body,
        grid=(NFF,),
        in_specs=[
            pl.BlockSpec((TOK, HIDDEN), lambda f: (0, 0)),
            pl.BlockSpec((E, HIDDEN), lambda f: (0, 0)),
            pl.BlockSpec((FFB, HIDDEN), lambda f: (f, 0)),
            pl.BlockSpec((FFB, HIDDEN), lambda f: (f, 0)),
            pl.BlockSpec((HIDDEN, FFB), lambda f: (0, f)),
        ],
        out_specs=[
            pl.BlockSpec((TOK, E), lambda f: (0, 0)),
            pl.BlockSpec((TOK, HIDDEN), lambda f: (0, 0)),
        ],
        out_shape=[
            jax.ShapeDtypeStruct((TOK, E), jnp.float32),
            jax.ShapeDtypeStruct((TOK, HIDDEN), jnp.float32),
        ],
        scratch_shapes=[pltpu.VMEM((TOK, HIDDEN), jnp.float32)],
        compiler_params=pltpu.CompilerParams(
            vmem_limit_bytes=128 * 1024 * 1024),
    )(hs, router_w, shared_gate_w, shared_up_w, shared_down_w)

    out = pl.pallas_call(
        _experts_body,
        grid=(E, NFF),
        in_specs=[
            pl.BlockSpec((TOK, HIDDEN), lambda e, f: (0, 0)),
            pl.BlockSpec((TOK, E), lambda e, f: (0, 0)),
            pl.BlockSpec((TOK, HIDDEN), lambda e, f: (0, 0)),
            pl.BlockSpec((1, FFB, HIDDEN), lambda e, f: (e, f, 0)),
            pl.BlockSpec((1, FFB, HIDDEN), lambda e, f: (e, f, 0)),
            pl.BlockSpec((1, HIDDEN, FFB), lambda e, f: (e, 0, f)),
        ],
        out_specs=pl.BlockSpec((TOK, HIDDEN), lambda e, f: (0, 0)),
        out_shape=jax.ShapeDtypeStruct((TOK, HIDDEN), jnp.float32),
        scratch_shapes=[pltpu.VMEM((TOK, HIDDEN), jnp.float32)],
        compiler_params=pltpu.CompilerParams(
            vmem_limit_bytes=128 * 1024 * 1024),
    )(hs, scores, base, gate_w, up_w, down_w)

    return out, scores
